# Initial kernel scaffold; baseline (speedup 1.0000x reference)
import jax
import jax.numpy as jnp
from jax.experimental import pallas as pl


def kernel(x, idx0, idx1, idx2, idx3, drow0, dcol0, dval0, drow1, dcol1, dval1, drow2, dcol2, dval2, drow3, dcol3, dval3, urow0, ucol0, uval0, urow1, ucol1, uval1, urow2, ucol2, uval2, urow3, ucol3, uval3, Wc0, bc0, Wc1, bc1, Wc2, bc2, Wc3, bc3, Wfc_en, bfc_en, Wfc_de, bfc_de, Wd0, bd0, Wd1, bd1, Wd2, bd2, Wd3, bd3, Wout, bout):
    raise NotImplementedError("write your pallas kernel here")


# speedup vs baseline: 9.4307x; 9.4307x over previous
import functools
import math

import jax
import jax.numpy as jnp
from jax import lax
from jax.experimental import pallas as pl
from jax.experimental.pallas import tpu as pltpu
from jax.experimental.pallas import tpu_sc as plsc

_NS = [50000, 12500, 3125, 800, 200]
_K = 9
_B = 4

_NC = 2
_NSUB = 16
_NW = _NC * _NSUB
_CHUNK = 128
_SUPER = 8
_SROWS = _CHUNK * _SUPER


@functools.partial(jax.jit, static_argnames=("T", "C"))
def _sc_gather(tab, idx_pad, *, T, C):
    nsup = idx_pad.shape[0] // _SUPER
    rem = T - (nsup - 1) * _SROWS
    niter = (nsup + _NW - 1) // _NW
    mesh = plsc.VectorSubcoreMesh(
        core_axis_name="c", subcore_axis_name="s",
        num_cores=_NC, num_subcores=_NSUB,
    )

    @functools.partial(
        pl.kernel,
        out_type=jax.ShapeDtypeStruct((T, C), jnp.float32),
        mesh=mesh,
        scratch_types=[
            pltpu.VMEM((_SUPER, _CHUNK), jnp.int32),
            pltpu.VMEM((_SROWS, C), jnp.float32),
            pltpu.SemaphoreType.DMA,
        ],
        compiler_params=pltpu.CompilerParams(use_tc_tiling_on_sc=False),
        interpret=False,
    )
    def k(tab_h, idx_h, out_h, idx_v, rows_v, sem):
        wid = lax.axis_index("s") * _NC + lax.axis_index("c")

        def body(i, carry):
            s = wid + i * _NW

            @pl.when(s < nsup)
            def _():
                base = s * _SROWS
                pltpu.sync_copy(idx_h.at[pl.ds(s * _SUPER, _SUPER)], idx_v)
                cps = [
                    pltpu.async_copy(
                        tab_h.at[idx_v.at[j]],
                        rows_v.at[pl.ds(j * _CHUNK, _CHUNK)],
                        sem,
                    )
                    for j in range(_SUPER)
                ]
                for cp in cps:
                    cp.wait()
                if rem == _SROWS:
                    pltpu.sync_copy(rows_v, out_h.at[pl.ds(base, _SROWS)])
                else:
                    @pl.when(s < nsup - 1)
                    def _():
                        pltpu.sync_copy(rows_v, out_h.at[pl.ds(base, _SROWS)])

                    @pl.when(s == nsup - 1)
                    def _():
                        pltpu.sync_copy(
                            rows_v.at[pl.ds(0, rem)],
                            out_h.at[pl.ds(base, rem)],
                        )

            return carry

        lax.fori_loop(0, niter, body, 0)

    return k(tab, idx_pad)


@functools.partial(jax.jit, static_argnames=("act",))
def _tc_linear(x, W, b, *, act):
    R, Kd = x.shape
    Co = W.shape[1]
    blk = min(2048, max(8, ((R + 7) // 8) * 8))
    grid = (R + blk - 1) // blk

    def body(x_ref, w_ref, b_ref, o_ref):
        acc = jnp.dot(x_ref[...], w_ref[...], preferred_element_type=jnp.float32)
        acc = acc + b_ref[...]
        if act:
            acc = jnp.where(acc > 0, acc, jnp.exp(jnp.minimum(acc, 0.0)) - 1.0)
        o_ref[...] = acc

    return pl.pallas_call(
        body,
        grid=(grid,),
        in_specs=[
            pl.BlockSpec((blk, Kd), lambda i: (i, 0)),
            pl.BlockSpec((Kd, Co), lambda i: (0, 0)),
            pl.BlockSpec((1, Co), lambda i: (0, 0)),
        ],
        out_specs=pl.BlockSpec((blk, Co), lambda i: (i, 0)),
        out_shape=jax.ShapeDtypeStruct((R, Co), jnp.float32),
        interpret=False,
    )(x, W, b)


def _tc_pool_reduce(rows3, val3):
    G, T2, C = rows3.shape
    blk = min(1024, T2)
    grid = (T2 + blk - 1) // blk

    def body(r_ref, v_ref, o_ref):
        acc = r_ref[0] * v_ref[0]
        for g in range(1, G):
            acc = acc + r_ref[g] * v_ref[g]
        o_ref[...] = acc

    return pl.pallas_call(
        body,
        grid=(grid,),
        in_specs=[
            pl.BlockSpec((G, blk, C), lambda i: (0, i, 0)),
            pl.BlockSpec((G, blk, 1), lambda i: (0, i, 0)),
        ],
        out_specs=pl.BlockSpec((blk, C), lambda i: (i, 0)),
        out_shape=jax.ShapeDtypeStruct((T2, C), jnp.float32),
        interpret=False,
    )(rows3, val3)


def _pad_idx(flat):
    T = flat.shape[0]
    Tp = ((T + _SROWS - 1) // _SROWS) * _SROWS
    if Tp != T:
        flat = jnp.concatenate([flat, jnp.zeros((Tp - T,), jnp.int32)])
    return flat.reshape(Tp // _CHUNK, _CHUNK)


def _conv_gather_idx(idx, R):
    offs = (jnp.arange(_B, dtype=jnp.int32) * R)[:, None]
    flat = (idx.reshape(1, -1).astype(jnp.int32) + offs).reshape(-1)
    return _pad_idx(flat)


def _pool_gather_idx(col, val, G, M, R):
    colr = col.reshape(M, G).T.astype(jnp.int32)
    offs = (jnp.arange(_B, dtype=jnp.int32) * R)[None, :, None]
    flat = (colr[:, None, :] + offs).reshape(-1)
    valr = val.reshape(M, G).T
    val3 = jnp.broadcast_to(valr[:, None, :], (G, _B, M)).reshape(G, _B * M, 1)
    return _pad_idx(flat), val3


def _spiral_conv(h2, gidx, N, C, W, b, *, act=True):
    g = _sc_gather(h2, gidx, T=_B * N * _K, C=C)
    return _tc_linear(g.reshape(_B * N, _K * C), W, b.reshape(1, -1), act=act)


def _pool(h2, pidx, val3, G, M, C):
    rows = _sc_gather(h2, pidx, T=G * _B * M, C=C)
    return _tc_pool_reduce(rows.reshape(G, _B * M, C), val3)


def kernel(x, idx0, idx1, idx2, idx3, drow0, dcol0, dval0, drow1, dcol1, dval1, drow2, dcol2, dval2, drow3, dcol3, dval3, urow0, ucol0, uval0, urow1, ucol1, uval1, urow2, ucol2, uval2, urow3, ucol3, uval3, Wc0, bc0, Wc1, bc1, Wc2, bc2, Wc3, bc3, Wfc_en, bfc_en, Wfc_de, bfc_de, Wd0, bd0, Wd1, bd1, Wd2, bd2, Wd3, bd3, Wout, bout):
    idxs = [idx0, idx1, idx2, idx3]
    dcols = [dcol0, dcol1, dcol2, dcol3]
    dvals = [dval0, dval1, dval2, dval3]
    ucols = [ucol0, ucol1, ucol2, ucol3]
    uvals = [uval0, uval1, uval2, uval3]
    Wens = [(Wc0, bc0), (Wc1, bc1), (Wc2, bc2), (Wc3, bc3)]
    Wdes = [(Wd0, bd0), (Wd1, bd1), (Wd2, bd2), (Wd3, bd3)]
    CH = [16, 16, 32, 64]
    enc_in = [8, 16, 16, 32]
    Wc0p = jnp.pad(Wc0.reshape(_K, 3, -1), ((0, 0), (0, 5), (0, 0))).reshape(_K * 8, -1)
    Wens = [(Wc0p, bc0)] + Wens[1:]

    gidx = [_conv_gather_idx(idxs[i], _NS[i]) for i in range(4)]
    dprep = [
        _pool_gather_idx(dcols[i], dvals[i], 4, _NS[i + 1], _NS[i])
        for i in range(4)
    ]
    uprep = [
        _pool_gather_idx(ucols[i], uvals[i], 3, _NS[i], _NS[i + 1])
        for i in range(4)
    ]

    h2 = jnp.pad(x.reshape(_B * _NS[0], 3), ((0, 0), (0, 5)))
    for i in range(4):
        h2 = _spiral_conv(h2, gidx[i], _NS[i], enc_in[i], Wens[i][0], Wens[i][1])
        h2 = _pool(h2, dprep[i][0], dprep[i][1], 4, _NS[i + 1], CH[i])

    z = _tc_linear(h2.reshape(_B, _NS[4] * CH[3]), Wfc_en, bfc_en.reshape(1, -1), act=False)
    h2 = _tc_linear(z, Wfc_de, bfc_de.reshape(1, -1), act=False).reshape(_B * _NS[4], CH[3])

    dec_in = [CH[3], CH[3], CH[2], CH[1]]
    for i in range(4):
        lvl = 3 - i
        h2 = _pool(h2, uprep[lvl][0], uprep[lvl][1], 3, _NS[lvl], dec_in[i])
        h2 = _spiral_conv(h2, gidx[lvl], _NS[lvl], dec_in[i], Wdes[i][0], Wdes[i][1])

    out = _spiral_conv(h2, gidx[0], _NS[0], CH[0], Wout, bout, act=False)
    return out.reshape(_B, _NS[0], 3)
